# baseline (speedup 1.0000x reference)
import functools

import jax
import jax.numpy as jnp
from jax import lax
from jax.experimental import pallas as pl
from jax.experimental.pallas import tpu as pltpu
from jax.experimental.pallas import tpu_sc as plsc

_THRESH = 0.02
_K = 16
_NC = 2
_NS = 16
_NW = _NC * _NS
_L = 16
_W = 128
_CROWS = 128
_NBUF = 5
_AHEAD = 2


def _dgather(table, idx):
    dn = lax.GatherDimensionNumbers(
        offset_dims=(), collapsed_slice_dims=(0,), start_index_map=(0,))
    return lax.gather(table, idx[:, None], dn, slice_sizes=(1,),
                      mode=lax.GatherScatterMode.PROMISE_IN_BOUNDS)


def _sc_body(p_hbm, cs_hbm, bs_hbm, o_hbm,
             bufs, cs_v, bs_v, ld_sems, st_sems):
    nrows = p_hbm.shape[0]
    per_w = nrows // _NW
    nchunks = per_w // _CROWS
    wid = lax.axis_index("s") * _NC + lax.axis_index("c")
    base = wid * per_w

    pltpu.sync_copy(cs_hbm, cs_v)
    pltpu.sync_copy(bs_hbm, bs_v)
    cs = cs_v[...]
    bs = bs_v[...]

    ld_handles = [None] * nchunks
    st_handles = [None] * nchunks

    def start_load(cj):
        b = cj % _NBUF
        ld_handles[cj] = pltpu.async_copy(
            p_hbm.at[pl.ds(base + cj * _CROWS, _CROWS)], bufs[b], ld_sems[b])

    for cj in range(min(_AHEAD + 1, nchunks)):
        start_load(cj)

    for ci in range(nchunks):
        b = ci % _NBUF
        nxt = ci + _AHEAD + 1
        if nxt < nchunks:
            prev = nxt - _NBUF
            if prev >= 0:
                st_handles[prev].wait()
            start_load(nxt)
        ld_handles[ci].wait()
        buf = bufs[b]

        @plsc.parallel_loop(0, _CROWS, step=1, unroll=2)
        def _row(r):
            for j in range(_W // _L):
                v = buf[r, pl.ds(j * _L, _L)]
                v = jnp.where(jnp.abs(v) > _THRESH, v, 0.0)
                lo = jnp.zeros((_L,), jnp.int32)
                for step in (8, 4, 2, 1):
                    mid = lo + step
                    t = _dgather(bs, mid)
                    lo = jnp.where(v >= t, mid, lo)
                buf[r, pl.ds(j * _L, _L)] = _dgather(cs, lo)

        st_handles[ci] = pltpu.async_copy(
            buf, o_hbm.at[pl.ds(base + ci * _CROWS, _CROWS)], st_sems[b])

    for ci in range(max(0, nchunks - _NBUF), nchunks):
        if st_handles[ci] is not None:
            st_handles[ci].wait()


def kernel(param, centroids):
    rows, cols = param.shape
    n = rows * cols
    cs = jnp.sort(centroids)
    mids = 0.5 * (cs[:-1] + cs[1:])
    bs = jnp.concatenate([jnp.full((1,), -jnp.inf, cs.dtype), mids])

    mesh = plsc.VectorSubcoreMesh(
        core_axis_name="c", subcore_axis_name="s",
        num_cores=_NC, num_subcores=_NS,
    )
    view = param.reshape(n // _W, _W)
    out = pl.kernel(
        _sc_body,
        out_type=jax.ShapeDtypeStruct((n // _W, _W), param.dtype),
        mesh=mesh,
        compiler_params=pltpu.CompilerParams(use_tc_tiling_on_sc=True),
        scratch_types=[
            tuple(pltpu.VMEM((_CROWS, _W), jnp.float32)
                  for _ in range(_NBUF)),
            pltpu.VMEM((_K,), jnp.float32),
            pltpu.VMEM((_K,), jnp.float32),
            tuple(pltpu.SemaphoreType.DMA for _ in range(_NBUF)),
            tuple(pltpu.SemaphoreType.DMA for _ in range(_NBUF)),
        ],
    )(view, cs, bs)
    return out.reshape(rows, cols)


# speedup vs baseline: 1.2816x; 1.2816x over previous
import functools

import jax
import jax.numpy as jnp
from jax import lax
from jax.experimental import pallas as pl
from jax.experimental.pallas import tpu as pltpu
from jax.experimental.pallas import tpu_sc as plsc

_THRESH = 0.02
_K = 16
_NC = 2
_NS = 16
_NW = _NC * _NS
_L = 16
_W = 128
_CROWS = 64
_NBUF = 5
_AHEAD = 2

_SC_VROWS = 6144
_TC_BLOCK_ROWS = 256


def _dgather(table, idx):
    dn = lax.GatherDimensionNumbers(
        offset_dims=(), collapsed_slice_dims=(0,), start_index_map=(0,))
    return lax.gather(table, idx[:, None], dn, slice_sizes=(1,),
                      mode=lax.GatherScatterMode.PROMISE_IN_BOUNDS)


def _sc_body(p_hbm, cs_hbm, bs_hbm, o_hbm,
             bufs, cs_v, bs_v, ld_sems, st_sems):
    nrows = p_hbm.shape[0]
    sc_rows = o_hbm.shape[0]
    start = nrows - sc_rows
    per_w = sc_rows // _NW
    nchunks = per_w // _CROWS
    wid = lax.axis_index("s") * _NC + lax.axis_index("c")
    base = start + wid * per_w
    obase = wid * per_w

    pltpu.sync_copy(cs_hbm, cs_v)
    pltpu.sync_copy(bs_hbm, bs_v)
    cs = cs_v[...]
    bs = bs_v[...]

    ld_handles = [None] * nchunks
    st_handles = [None] * nchunks

    def start_load(cj):
        b = cj % _NBUF
        ld_handles[cj] = pltpu.async_copy(
            p_hbm.at[pl.ds(base + cj * _CROWS, _CROWS)], bufs[b], ld_sems[b])

    for cj in range(min(_AHEAD + 1, nchunks)):
        start_load(cj)

    for ci in range(nchunks):
        b = ci % _NBUF
        nxt = ci + _AHEAD + 1
        if nxt < nchunks:
            prev = nxt - _NBUF
            if prev >= 0:
                st_handles[prev].wait()
            start_load(nxt)
        ld_handles[ci].wait()
        buf = bufs[b]

        @plsc.parallel_loop(0, _CROWS, step=1, unroll=2)
        def _row(r):
            for j in range(_W // _L):
                v = buf[r, pl.ds(j * _L, _L)]
                v = jnp.where(jnp.abs(v) > _THRESH, v, 0.0)
                lo = jnp.zeros((_L,), jnp.int32)
                for step in (8, 4, 2, 1):
                    mid = lo + step
                    t = _dgather(bs, mid)
                    lo = jnp.where(v >= t, mid, lo)
                buf[r, pl.ds(j * _L, _L)] = _dgather(cs, lo)

        st_handles[ci] = pltpu.async_copy(
            buf, o_hbm.at[pl.ds(obase + ci * _CROWS, _CROWS)], st_sems[b])

    for ci in range(max(0, nchunks - _NBUF), nchunks):
        if st_handles[ci] is not None:
            st_handles[ci].wait()


def _tc_quant_kernel(cs_ref, b_ref, p_ref, o_ref):
    v = p_ref[...]
    keep = jnp.abs(v) > _THRESH
    res = jnp.full(v.shape, cs_ref[0], v.dtype)
    for i in range(_K - 1):
        res = jnp.where(v > b_ref[i], cs_ref[i + 1], res)
    o_ref[...] = jnp.where(keep, res, cs_ref[_K])


def kernel(param, centroids):
    rows, cols = param.shape
    n = rows * cols
    cs = jnp.sort(centroids)
    mids = 0.5 * (cs[:-1] + cs[1:])
    bs = jnp.concatenate([jnp.full((1,), -jnp.inf, cs.dtype), mids])
    zidx = jnp.sum((mids < 0.0).astype(jnp.int32))
    cs_ext = jnp.concatenate([cs, cs[zidx][None]])

    sc_rows = _SC_VROWS * _W // cols
    rows_tc = rows - sc_rows

    mesh = plsc.VectorSubcoreMesh(
        core_axis_name="c", subcore_axis_name="s",
        num_cores=_NC, num_subcores=_NS,
    )
    view = param.reshape(n // _W, _W)
    sc_out = pl.kernel(
        _sc_body,
        out_type=jax.ShapeDtypeStruct((_SC_VROWS, _W), param.dtype),
        mesh=mesh,
        compiler_params=pltpu.CompilerParams(use_tc_tiling_on_sc=True),
        scratch_types=[
            tuple(pltpu.VMEM((_CROWS, _W), jnp.float32)
                  for _ in range(_NBUF)),
            pltpu.VMEM((_K,), jnp.float32),
            pltpu.VMEM((_K,), jnp.float32),
            tuple(pltpu.SemaphoreType.DMA for _ in range(_NBUF)),
            tuple(pltpu.SemaphoreType.DMA for _ in range(_NBUF)),
        ],
    )(view, cs, bs)

    tc_out = pl.pallas_call(
        _tc_quant_kernel,
        grid=(rows_tc // _TC_BLOCK_ROWS,),
        in_specs=[
            pl.BlockSpec(memory_space=pltpu.SMEM),
            pl.BlockSpec(memory_space=pltpu.SMEM),
            pl.BlockSpec((_TC_BLOCK_ROWS, cols), lambda i: (i, 0)),
        ],
        out_specs=pl.BlockSpec((_TC_BLOCK_ROWS, cols), lambda i: (i, 0)),
        out_shape=jax.ShapeDtypeStruct((rows_tc, cols), param.dtype),
    )(cs_ext, mids, param)

    return jnp.concatenate(
        [tc_out, sc_out.reshape(sc_rows, cols)], axis=0)


# speedup vs baseline: 1.2838x; 1.0017x over previous
import functools

import jax
import jax.numpy as jnp
from jax import lax
from jax.experimental import pallas as pl
from jax.experimental.pallas import tpu as pltpu
from jax.experimental.pallas import tpu_sc as plsc

_THRESH = 0.02
_K = 16
_NC = 2
_NS = 16
_NW = _NC * _NS
_L = 16
_W = 128
_CROWS = 64
_NBUF = 5
_AHEAD = 2

_SC_VROWS = 6144
_TC_BLOCK_ROWS = 256


def _dgather(table, idx):
    dn = lax.GatherDimensionNumbers(
        offset_dims=(), collapsed_slice_dims=(0,), start_index_map=(0,))
    return lax.gather(table, idx[:, None], dn, slice_sizes=(1,),
                      mode=lax.GatherScatterMode.PROMISE_IN_BOUNDS)


def _sc_body(p_hbm, cs_hbm, bs_hbm, o_hbm,
             bufs, cs_v, bs_v, ld_sems, st_sems):
    nrows = p_hbm.shape[0]
    sc_rows = o_hbm.shape[0]
    start = nrows - sc_rows
    per_w = sc_rows // _NW
    nchunks = per_w // _CROWS
    wid = lax.axis_index("s") * _NC + lax.axis_index("c")
    base = start + wid * per_w
    obase = wid * per_w

    pltpu.sync_copy(cs_hbm, cs_v)
    pltpu.sync_copy(bs_hbm, bs_v)
    cs = cs_v[...]
    bs = bs_v[...]

    ld_handles = [None] * nchunks
    st_handles = [None] * nchunks

    def start_load(cj):
        b = cj % _NBUF
        ld_handles[cj] = pltpu.async_copy(
            p_hbm.at[pl.ds(base + cj * _CROWS, _CROWS)], bufs[b], ld_sems[b])

    for cj in range(min(_AHEAD + 1, nchunks)):
        start_load(cj)

    for ci in range(nchunks):
        b = ci % _NBUF
        nxt = ci + _AHEAD + 1
        if nxt < nchunks:
            prev = nxt - _NBUF
            if prev >= 0:
                st_handles[prev].wait()
            start_load(nxt)
        ld_handles[ci].wait()
        buf = bufs[b]

        @plsc.parallel_loop(0, _CROWS, step=1, unroll=2)
        def _row(r):
            for j in range(_W // _L):
                v = buf[r, pl.ds(j * _L, _L)]
                v = jnp.where(jnp.abs(v) > _THRESH, v, 0.0)
                lo = jnp.zeros((_L,), jnp.int32)
                for step in (8, 4, 2, 1):
                    mid = lo + step
                    t = _dgather(bs, mid)
                    lo = jnp.where(v >= t, mid, lo)
                buf[r, pl.ds(j * _L, _L)] = _dgather(cs, lo)

        st_handles[ci] = pltpu.async_copy(
            buf, o_hbm.at[pl.ds(obase + ci * _CROWS, _CROWS)], st_sems[b])

    for ci in range(max(0, nchunks - _NBUF), nchunks):
        if st_handles[ci] is not None:
            st_handles[ci].wait()


def _tc_quant_kernel(cs_ref, b_ref, p_ref, o_ref):
    v = p_ref[...]
    keep = jnp.abs(v) > _THRESH
    res = jnp.full(v.shape, cs_ref[0], v.dtype)
    for i in range(_K - 1):
        res = jnp.where(v > b_ref[i], cs_ref[i + 1], res)
    o_ref[...] = jnp.where(keep, res, cs_ref[_K])


def kernel(param, centroids):
    rows, cols = param.shape
    n = rows * cols
    cs = jnp.sort(centroids)
    mids = 0.5 * (cs[:-1] + cs[1:])
    bs = jnp.concatenate([jnp.full((1,), -jnp.inf, cs.dtype), mids])
    zidx = jnp.sum((mids < 0.0).astype(jnp.int32))
    cs_ext = jnp.concatenate([cs, cs[zidx][None]])

    sc_rows = _SC_VROWS * _W // cols
    rows_tc = rows - sc_rows

    mesh = plsc.VectorSubcoreMesh(
        core_axis_name="c", subcore_axis_name="s",
        num_cores=_NC, num_subcores=_NS,
    )
    view = param.reshape(n // _W, _W)
    sc_out = pl.kernel(
        _sc_body,
        out_type=jax.ShapeDtypeStruct((_SC_VROWS, _W), param.dtype),
        mesh=mesh,
        scratch_types=[
            tuple(pltpu.VMEM((_CROWS, _W), jnp.float32)
                  for _ in range(_NBUF)),
            pltpu.VMEM((_K,), jnp.float32),
            pltpu.VMEM((_K,), jnp.float32),
            tuple(pltpu.SemaphoreType.DMA for _ in range(_NBUF)),
            tuple(pltpu.SemaphoreType.DMA for _ in range(_NBUF)),
        ],
    )(view, cs, bs)

    tc_out = pl.pallas_call(
        _tc_quant_kernel,
        grid=(rows_tc // _TC_BLOCK_ROWS,),
        in_specs=[
            pl.BlockSpec(memory_space=pltpu.SMEM),
            pl.BlockSpec(memory_space=pltpu.SMEM),
            pl.BlockSpec((_TC_BLOCK_ROWS, cols), lambda i: (i, 0)),
        ],
        out_specs=pl.BlockSpec((_TC_BLOCK_ROWS, cols), lambda i: (i, 0)),
        out_shape=jax.ShapeDtypeStruct((rows_tc, cols), param.dtype),
    )(cs_ext, mids, param)

    return jnp.concatenate(
        [tc_out, sc_out.reshape(sc_rows, cols)], axis=0)


# speedup vs baseline: 1.6738x; 1.3038x over previous
import functools

import jax
import jax.numpy as jnp
from jax import lax
from jax.experimental import pallas as pl
from jax.experimental.pallas import tpu as pltpu
from jax.experimental.pallas import tpu_sc as plsc

_THRESH = 0.02
_K = 16
_NC = 2
_NS = 16
_NW = _NC * _NS
_L = 16

_SC_ROWS = 512
_CROWS = 8
_NBUF = 2
_AHEAD = 1
_TC_BLOCK_ROWS = 128


def _dgather(table, idx):
    dn = lax.GatherDimensionNumbers(
        offset_dims=(), collapsed_slice_dims=(0,), start_index_map=(0,))
    return lax.gather(table, idx[:, None], dn, slice_sizes=(1,),
                      mode=lax.GatherScatterMode.PROMISE_IN_BOUNDS)


def _sc_body(p_hbm, cs_hbm, bs_hbm, o_hbm,
             bufs, cs_v, bs_v, ld_sems, st_sems):
    nrows = p_hbm.shape[0]
    ncols = p_hbm.shape[1]
    sc_rows = o_hbm.shape[0]
    start = nrows - sc_rows
    per_w = sc_rows // _NW
    nchunks = per_w // _CROWS
    wid = lax.axis_index("s") * _NC + lax.axis_index("c")
    base = start + wid * per_w
    obase = wid * per_w

    pltpu.sync_copy(cs_hbm, cs_v)
    pltpu.sync_copy(bs_hbm, bs_v)
    cs = cs_v[...]
    bs = bs_v[...]

    ld_handles = [None] * nchunks
    st_handles = [None] * nchunks

    def start_load(cj):
        b = cj % _NBUF
        ld_handles[cj] = pltpu.async_copy(
            p_hbm.at[pl.ds(base + cj * _CROWS, _CROWS)], bufs[b], ld_sems[b])

    for cj in range(min(_AHEAD + 1, nchunks)):
        start_load(cj)

    for ci in range(nchunks):
        b = ci % _NBUF
        nxt = ci + _AHEAD + 1
        if nxt < nchunks:
            prev = nxt - _NBUF
            if prev >= 0:
                st_handles[prev].wait()
            start_load(nxt)
        ld_handles[ci].wait()
        buf = bufs[b]

        @plsc.parallel_loop(0, ncols, step=_L, unroll=2)
        def _col(c):
            for r in range(_CROWS):
                v = buf[r, pl.ds(c, _L)]
                v = jnp.where(jnp.abs(v) > _THRESH, v, 0.0)
                lo = jnp.zeros((_L,), jnp.int32)
                for step in (8, 4, 2, 1):
                    mid = lo + step
                    t = _dgather(bs, mid)
                    lo = jnp.where(v >= t, mid, lo)
                buf[r, pl.ds(c, _L)] = _dgather(cs, lo)

        st_handles[ci] = pltpu.async_copy(
            buf, o_hbm.at[pl.ds(obase + ci * _CROWS, _CROWS)], st_sems[b])

    for ci in range(max(0, nchunks - _NBUF), nchunks):
        if st_handles[ci] is not None:
            st_handles[ci].wait()


def _tc_quant_kernel(cs_ref, b_ref, p_ref, o_ref):
    v = p_ref[...]
    keep = jnp.abs(v) > _THRESH
    res = jnp.full(v.shape, cs_ref[0], v.dtype)
    for i in range(_K - 1):
        res = jnp.where(v > b_ref[i], cs_ref[i + 1], res)
    o_ref[...] = jnp.where(keep, res, cs_ref[_K])


def kernel(param, centroids):
    rows, cols = param.shape
    cs = jnp.sort(centroids)
    mids = 0.5 * (cs[:-1] + cs[1:])
    bs = jnp.concatenate([jnp.full((1,), -jnp.inf, cs.dtype), mids])
    zidx = jnp.sum((mids < 0.0).astype(jnp.int32))
    cs_ext = jnp.concatenate([cs, cs[zidx][None]])

    rows_tc = rows - _SC_ROWS

    mesh = plsc.VectorSubcoreMesh(
        core_axis_name="c", subcore_axis_name="s",
        num_cores=_NC, num_subcores=_NS,
    )
    sc_out = pl.kernel(
        _sc_body,
        out_type=jax.ShapeDtypeStruct((_SC_ROWS, cols), param.dtype),
        mesh=mesh,
        compiler_params=pltpu.CompilerParams(use_tc_tiling_on_sc=True),
        scratch_types=[
            tuple(pltpu.VMEM((_CROWS, cols), jnp.float32)
                  for _ in range(_NBUF)),
            pltpu.VMEM((_K,), jnp.float32),
            pltpu.VMEM((_K,), jnp.float32),
            tuple(pltpu.SemaphoreType.DMA for _ in range(_NBUF)),
            tuple(pltpu.SemaphoreType.DMA for _ in range(_NBUF)),
        ],
    )(param, cs, bs)

    tc_out = pl.pallas_call(
        _tc_quant_kernel,
        grid=(rows_tc // _TC_BLOCK_ROWS,),
        in_specs=[
            pl.BlockSpec(memory_space=pltpu.SMEM),
            pl.BlockSpec(memory_space=pltpu.SMEM),
            pl.BlockSpec((_TC_BLOCK_ROWS, cols), lambda i: (i, 0)),
        ],
        out_specs=pl.BlockSpec((_TC_BLOCK_ROWS, cols), lambda i: (i, 0)),
        out_shape=jax.ShapeDtypeStruct((rows_tc, cols), param.dtype),
    )(cs_ext, mids, param)

    return jnp.concatenate([tc_out, sc_out], axis=0)


# speedup vs baseline: 1.9881x; 1.1877x over previous
import functools

import jax
import jax.numpy as jnp
from jax import lax
from jax.experimental import pallas as pl
from jax.experimental.pallas import tpu as pltpu
from jax.experimental.pallas import tpu_sc as plsc

_THRESH = 0.02
_K = 16
_NC = 2
_NS = 16
_NW = _NC * _NS
_L = 16

_SC_ROWS = 512
_CROWS = 8
_NBUF = 2
_AHEAD = 1
_TC_BLOCK_ROWS = 128


def _dgather(table, idx):
    dn = lax.GatherDimensionNumbers(
        offset_dims=(), collapsed_slice_dims=(0,), start_index_map=(0,))
    return lax.gather(table, idx[:, None], dn, slice_sizes=(1,),
                      mode=lax.GatherScatterMode.PROMISE_IN_BOUNDS)


def _sc_body(p_hbm, cs_hbm, bs_hbm, o_hbm,
             bufs, cs_v, bs_v, ld_sems, st_sems):
    nrows = p_hbm.shape[0]
    ncols = p_hbm.shape[1]
    sc_rows = o_hbm.shape[0]
    start = nrows - sc_rows
    per_w = sc_rows // _NW
    nchunks = per_w // _CROWS
    wid = lax.axis_index("s") * _NC + lax.axis_index("c")
    base = start + wid * per_w
    obase = wid * per_w

    pltpu.sync_copy(cs_hbm, cs_v)
    pltpu.sync_copy(bs_hbm, bs_v)
    cs = cs_v[...]
    bs = bs_v[...]

    ld_handles = [None] * nchunks
    st_handles = [None] * nchunks

    def start_load(cj):
        b = cj % _NBUF
        ld_handles[cj] = pltpu.async_copy(
            p_hbm.at[pl.ds(base + cj * _CROWS, _CROWS)], bufs[b], ld_sems[b])

    for cj in range(min(_AHEAD + 1, nchunks)):
        start_load(cj)

    for ci in range(nchunks):
        b = ci % _NBUF
        nxt = ci + _AHEAD + 1
        if nxt < nchunks:
            prev = nxt - _NBUF
            if prev >= 0:
                st_handles[prev].wait()
            start_load(nxt)
        ld_handles[ci].wait()
        buf = bufs[b]

        @plsc.parallel_loop(0, ncols, step=_L, unroll=2)
        def _col(c):
            for r in range(_CROWS):
                v = buf[r, pl.ds(c, _L)]
                v = jnp.where(jnp.abs(v) > _THRESH, v, 0.0)
                lo = jnp.zeros((_L,), jnp.int32)
                for step in (8, 4, 2, 1):
                    mid = lo + step
                    t = _dgather(bs, mid)
                    lo = jnp.where(v >= t, mid, lo)
                buf[r, pl.ds(c, _L)] = _dgather(cs, lo)

        st_handles[ci] = pltpu.async_copy(
            buf, o_hbm.at[pl.ds(obase + ci * _CROWS, _CROWS)], st_sems[b])

    for ci in range(max(0, nchunks - _NBUF), nchunks):
        if st_handles[ci] is not None:
            st_handles[ci].wait()


def _tc_quant_kernel(cs_ref, b_ref, p_ref, o_ref):
    v = p_ref[...]
    keep = jnp.abs(v) > _THRESH
    res = jnp.full(v.shape, cs_ref[0], v.dtype)
    for i in range(_K - 1):
        res = jnp.where(v > b_ref[i], cs_ref[i + 1], res)
    o_ref[...] = jnp.where(keep, res, cs_ref[_K])


def kernel(param, centroids):
    rows, cols = param.shape
    cs = jnp.sort(centroids)
    mids = 0.5 * (cs[:-1] + cs[1:])
    bs = jnp.concatenate([jnp.full((1,), -jnp.inf, cs.dtype), mids])
    zidx = jnp.sum((mids < 0.0).astype(jnp.int32))
    cs_ext = jnp.concatenate([cs, cs[zidx][None]])

    rows_tc = rows - _SC_ROWS

    mesh = plsc.VectorSubcoreMesh(
        core_axis_name="c", subcore_axis_name="s",
        num_cores=_NC, num_subcores=_NS,
    )
    sc_out = pl.kernel(
        _sc_body,
        out_type=jax.ShapeDtypeStruct((_SC_ROWS, cols), param.dtype),
        mesh=mesh,
        compiler_params=pltpu.CompilerParams(use_tc_tiling_on_sc=True),
        scratch_types=[
            tuple(pltpu.VMEM((_CROWS, cols), jnp.float32)
                  for _ in range(_NBUF)),
            pltpu.VMEM((_K,), jnp.float32),
            pltpu.VMEM((_K,), jnp.float32),
            tuple(pltpu.SemaphoreType.DMA for _ in range(_NBUF)),
            tuple(pltpu.SemaphoreType.DMA for _ in range(_NBUF)),
        ],
    )(param, cs, bs)

    tc_out = pl.pallas_call(
        _tc_quant_kernel,
        grid=(rows_tc // _TC_BLOCK_ROWS,),
        in_specs=[
            pl.BlockSpec(memory_space=pltpu.SMEM),
            pl.BlockSpec(memory_space=pltpu.SMEM),
            pl.BlockSpec((_TC_BLOCK_ROWS, cols), lambda i: (i, 0)),
        ],
        out_specs=pl.BlockSpec((_TC_BLOCK_ROWS, cols), lambda i: (i, 0)),
        out_shape=jax.ShapeDtypeStruct((rows, cols), param.dtype),
    )(cs_ext, mids, param)

    return lax.dynamic_update_slice(tc_out, sc_out, (rows_tc, 0))


# speedup vs baseline: 2.2153x; 1.1143x over previous
import functools

import jax
import jax.numpy as jnp
from jax import lax
from jax.experimental import pallas as pl
from jax.experimental.pallas import tpu as pltpu
from jax.experimental.pallas import tpu_sc as plsc

_THRESH = 0.02
_K = 16
_NC = 2
_NS = 16
_NW = _NC * _NS
_L = 16

_SC_ROWS = 512
_CROWS = 8
_NBUF = 2
_AHEAD = 1
_TC_BLOCK_ROWS = 128


def _dgather(table, idx):
    dn = lax.GatherDimensionNumbers(
        offset_dims=(), collapsed_slice_dims=(0,), start_index_map=(0,))
    return lax.gather(table, idx[:, None], dn, slice_sizes=(1,),
                      mode=lax.GatherScatterMode.PROMISE_IN_BOUNDS)


def _sc_body(p_hbm, cs_hbm, bs_hbm, o_hbm,
             bufs, cs_v, bs_v, ld_sems, st_sems):
    nrows = p_hbm.shape[0]
    ncols = p_hbm.shape[1]
    sc_rows = o_hbm.shape[0]
    start = nrows - sc_rows
    per_w = sc_rows // _NW
    nchunks = per_w // _CROWS
    wid = lax.axis_index("s") * _NC + lax.axis_index("c")
    base = start + wid * per_w
    obase = wid * per_w

    pltpu.sync_copy(cs_hbm, cs_v)
    pltpu.sync_copy(bs_hbm, bs_v)
    cs = cs_v[...]
    bs = bs_v[...]

    ld_handles = [None] * nchunks
    st_handles = [None] * nchunks

    def start_load(cj):
        b = cj % _NBUF
        ld_handles[cj] = pltpu.async_copy(
            p_hbm.at[pl.ds(base + cj * _CROWS, _CROWS)], bufs[b], ld_sems[b])

    for cj in range(min(_AHEAD + 1, nchunks)):
        start_load(cj)

    for ci in range(nchunks):
        b = ci % _NBUF
        nxt = ci + _AHEAD + 1
        if nxt < nchunks:
            prev = nxt - _NBUF
            if prev >= 0:
                st_handles[prev].wait()
            start_load(nxt)
        ld_handles[ci].wait()
        buf = bufs[b]

        @plsc.parallel_loop(0, ncols, step=_L, unroll=2)
        def _col(c):
            for r in range(_CROWS):
                v = buf[r, pl.ds(c, _L)]
                v = jnp.where(jnp.abs(v) > _THRESH, v, 0.0)
                lo = jnp.zeros((_L,), jnp.int32)
                for step in (8, 4, 2, 1):
                    mid = lo + step
                    t = _dgather(bs, mid)
                    lo = jnp.where(v >= t, mid, lo)
                buf[r, pl.ds(c, _L)] = _dgather(cs, lo)

        st_handles[ci] = pltpu.async_copy(
            buf, o_hbm.at[pl.ds(obase + ci * _CROWS, _CROWS)], st_sems[b])

    for ci in range(max(0, nchunks - _NBUF), nchunks):
        if st_handles[ci] is not None:
            st_handles[ci].wait()


def _tc_quant_kernel(c_ref, p_ref, o_ref, cs_ref, b_ref):
    @pl.when(pl.program_id(0) == 0)
    def _init():
        for i in range(_K):
            cs_ref[i] = c_ref[i]
        for i in range(_K):
            for j in range(_K - 1 - i):
                a = cs_ref[j]
                b = cs_ref[j + 1]
                cs_ref[j] = jnp.minimum(a, b)
                cs_ref[j + 1] = jnp.maximum(a, b)
        cz = cs_ref[0]
        for j in range(_K - 1):
            m = 0.5 * (cs_ref[j] + cs_ref[j + 1])
            b_ref[j] = m
            cz = jnp.where(m < 0.0, cs_ref[j + 1], cz)
        cs_ref[_K] = cz

    v = p_ref[...]
    keep = jnp.abs(v) > _THRESH
    res = jnp.full(v.shape, cs_ref[0], v.dtype)
    for i in range(_K - 1):
        res = jnp.where(v > b_ref[i], cs_ref[i + 1], res)
    o_ref[...] = jnp.where(keep, res, cs_ref[_K])


def kernel(param, centroids):
    rows, cols = param.shape
    cs = jnp.sort(centroids)
    mids = 0.5 * (cs[:-1] + cs[1:])
    bs = jnp.concatenate([jnp.full((1,), -jnp.inf, cs.dtype), mids])

    rows_tc = rows - _SC_ROWS

    mesh = plsc.VectorSubcoreMesh(
        core_axis_name="c", subcore_axis_name="s",
        num_cores=_NC, num_subcores=_NS,
    )
    sc_out = pl.kernel(
        _sc_body,
        out_type=jax.ShapeDtypeStruct((_SC_ROWS, cols), param.dtype),
        mesh=mesh,
        compiler_params=pltpu.CompilerParams(use_tc_tiling_on_sc=True),
        scratch_types=[
            tuple(pltpu.VMEM((_CROWS, cols), jnp.float32)
                  for _ in range(_NBUF)),
            pltpu.VMEM((_K,), jnp.float32),
            pltpu.VMEM((_K,), jnp.float32),
            tuple(pltpu.SemaphoreType.DMA for _ in range(_NBUF)),
            tuple(pltpu.SemaphoreType.DMA for _ in range(_NBUF)),
        ],
    )(param, cs, bs)

    tc_out = pl.pallas_call(
        _tc_quant_kernel,
        grid=(rows_tc // _TC_BLOCK_ROWS,),
        in_specs=[
            pl.BlockSpec(memory_space=pltpu.SMEM),
            pl.BlockSpec((_TC_BLOCK_ROWS, cols), lambda i: (i, 0)),
        ],
        out_specs=pl.BlockSpec((_TC_BLOCK_ROWS, cols), lambda i: (i, 0)),
        out_shape=jax.ShapeDtypeStruct((rows, cols), param.dtype),
        scratch_shapes=[
            pltpu.SMEM((_K + 1,), jnp.float32),
            pltpu.SMEM((_K - 1,), jnp.float32),
        ],
    )(centroids, param)

    return lax.dynamic_update_slice(tc_out, sc_out, (rows_tc, 0))
